# Initial kernel scaffold; baseline (speedup 1.0000x reference)
import jax
import jax.numpy as jnp
from jax.experimental import pallas as pl


def kernel(x, memory, pos_bias):
    raise NotImplementedError("write your pallas kernel here")


# speedup vs baseline: 139.0123x; 139.0123x over previous
import jax
import jax.numpy as jnp
from jax.experimental import pallas as pl


def _fused(x_ref, mem_ref, pb_ref, out_ref):
    x = x_ref[...]
    mem = mem_ref[...]
    pb = pb_ref[...]
    _, d = x.shape

    z1 = jnp.zeros((1, d), x.dtype)
    z2 = jnp.zeros((2, d), x.dtype)
    x2 = x
    x1 = jnp.concatenate([z1, x[:-1]], axis=0)
    x0 = jnp.concatenate([z2, x[:-2]], axis=0)

    def score(xs, n):
        w = mem[n]
        s = jax.lax.dot_general(xs, w, (((1,), (1,)), ((), ())),
                                preferred_element_type=jnp.float32)
        return s + pb[n][None, :]

    s0 = score(x0, 0)
    s1 = score(x1, 1)
    s2 = score(x2, 2)

    o0 = (s0 >= s1) & (s0 >= s2)
    o1 = jnp.logical_not(o0) & (s1 >= s2)
    o2 = jnp.logical_not(o0) & jnp.logical_not(o1)

    def combine(o, n):
        f = o.astype(jnp.float32)
        return jax.lax.dot_general(f, mem[n], (((1,), (0,)), ((), ())),
                                   preferred_element_type=jnp.float32)

    out_ref[...] = combine(o0, 0) + combine(o1, 1) + combine(o2, 2)


def kernel(x, memory, pos_bias):
    b, l, d = x.shape
    mem_t = memory.transpose(1, 0, 2)
    pb_t = pos_bias.T
    out = pl.pallas_call(
        _fused,
        out_shape=jax.ShapeDtypeStruct((l, d), jnp.float32),
    )(x[0], mem_t, pb_t)
    return out[None]


# speedup vs baseline: 146.9301x; 1.0570x over previous
import jax
import jax.numpy as jnp
from jax.experimental import pallas as pl
from jax.experimental.pallas import tpu as pltpu

_TILE = 256


def _fused(x_ref, w_ref, mem_ref, pb_ref, out_ref, carry_ref):
    i = pl.program_id(0)
    t = x_ref.shape[0]

    @pl.when(i == 0)
    def _():
        carry_ref[...] = jnp.zeros_like(carry_ref)

    y = jax.lax.dot_general(x_ref[...], w_ref[...], (((1,), (0,)), ((), ())),
                            preferred_element_type=jnp.float32)
    pb = pb_ref[...]
    prev = carry_ref[...]
    full01 = jnp.concatenate([prev, y[:, 0:128]], axis=0)
    carry_ref[...] = y[t - 2:t, 0:128]

    s0 = full01[0:t, 0:64] + pb[0][None, :]
    s1 = full01[1:t + 1, 64:128] + pb[1][None, :]
    s2 = y[:, 128:192] + pb[2][None, :]

    o0 = (s0 >= s1) & (s0 >= s2)
    o1 = jnp.logical_not(o0) & (s1 >= s2)
    f0 = o0.astype(jnp.float32)
    f1 = o1.astype(jnp.float32)

    mem = mem_ref[...]
    d0 = mem[0] - mem[2]
    d1 = mem[1] - mem[2]
    base = jnp.sum(mem[2], axis=0)[None, :]

    out = jax.lax.dot_general(f0, d0, (((1,), (0,)), ((), ())),
                              preferred_element_type=jnp.float32)
    out += jax.lax.dot_general(f1, d1, (((1,), (0,)), ((), ())),
                               preferred_element_type=jnp.float32)
    out_ref[...] = out + base


def kernel(x, memory, pos_bias):
    b, l, d = x.shape
    m, n = pos_bias.shape
    w = memory.transpose(1, 0, 2).reshape(n * m, d).T
    mem_t = memory.transpose(1, 0, 2)
    pb_t = pos_bias.T
    grid = (l // _TILE,)
    out = pl.pallas_call(
        _fused,
        grid=grid,
        in_specs=[
            pl.BlockSpec((_TILE, d), lambda i: (i, 0)),
            pl.BlockSpec((d, n * m), lambda i: (0, 0)),
            pl.BlockSpec((n, m, d), lambda i: (0, 0, 0)),
            pl.BlockSpec((n, m), lambda i: (0, 0)),
        ],
        out_specs=pl.BlockSpec((_TILE, d), lambda i: (i, 0)),
        scratch_shapes=[pltpu.VMEM((2, 2 * m), jnp.float32)],
        out_shape=jax.ShapeDtypeStruct((l, d), jnp.float32),
    )(x[0], w, mem_t, pb_t)
    return out[None]


# speedup vs baseline: 178.3087x; 1.2136x over previous
import jax
import jax.numpy as jnp
from jax.experimental import pallas as pl
from jax.experimental.pallas import tpu as pltpu

_TILE = 512


def _fused(x_ref, w_ref, mem_ref, pb_ref, out_ref, carry_ref):
    i = pl.program_id(0)
    t = x_ref.shape[0]

    @pl.when(i == 0)
    def _():
        carry_ref[...] = jnp.zeros_like(carry_ref)

    y = jax.lax.dot_general(x_ref[...], w_ref[...], (((1,), (0,)), ((), ())),
                            preferred_element_type=jnp.float32)
    pb = pb_ref[...]
    prev = carry_ref[...]
    full01 = jnp.concatenate([prev, y[:, 0:128]], axis=0)
    carry_ref[...] = y[t - 2:t, 0:128]

    s0 = full01[0:t, 0:64] + pb[0][None, :]
    s1 = full01[1:t + 1, 64:128] + pb[1][None, :]
    s2 = y[:, 128:192] + pb[2][None, :]

    o0 = (s0 >= s1) & (s0 >= s2)
    o1 = jnp.logical_not(o0) & (s1 >= s2)
    f0 = o0.astype(jnp.bfloat16)
    f1 = o1.astype(jnp.bfloat16)

    mem = mem_ref[...]
    d0 = (mem[0] - mem[2]).astype(jnp.bfloat16)
    d1 = (mem[1] - mem[2]).astype(jnp.bfloat16)
    base = jnp.sum(mem[2], axis=0)[None, :]

    out = jax.lax.dot_general(f0, d0, (((1,), (0,)), ((), ())),
                              preferred_element_type=jnp.float32)
    out += jax.lax.dot_general(f1, d1, (((1,), (0,)), ((), ())),
                               preferred_element_type=jnp.float32)
    out_ref[...] = out + base


def kernel(x, memory, pos_bias):
    b, l, d = x.shape
    m, n = pos_bias.shape
    w = memory.transpose(1, 0, 2).reshape(n * m, d).T
    mem_t = memory.transpose(1, 0, 2)
    pb_t = pos_bias.T
    grid = (l // _TILE,)
    out = pl.pallas_call(
        _fused,
        grid=grid,
        in_specs=[
            pl.BlockSpec((_TILE, d), lambda i: (i, 0)),
            pl.BlockSpec((d, n * m), lambda i: (0, 0)),
            pl.BlockSpec((n, m, d), lambda i: (0, 0, 0)),
            pl.BlockSpec((n, m), lambda i: (0, 0)),
        ],
        out_specs=pl.BlockSpec((_TILE, d), lambda i: (i, 0)),
        scratch_shapes=[pltpu.VMEM((2, 2 * m), jnp.float32)],
        out_shape=jax.ShapeDtypeStruct((l, d), jnp.float32),
    )(x[0], w, mem_t, pb_t)
    return out[None]


# speedup vs baseline: 185.4464x; 1.0400x over previous
import jax
import jax.numpy as jnp
from jax.experimental import pallas as pl
from jax.experimental.pallas import tpu as pltpu

_TILE = 1024


def _fused(x_ref, w_ref, mem_ref, pb_ref, out_ref, carry_ref):
    i = pl.program_id(0)
    t = x_ref.shape[0]

    @pl.when(i == 0)
    def _():
        carry_ref[...] = jnp.zeros_like(carry_ref)

    y = jax.lax.dot_general(x_ref[...], w_ref[...], (((1,), (0,)), ((), ())),
                            preferred_element_type=jnp.float32)
    pb = pb_ref[...]
    prev = carry_ref[...]
    full01 = jnp.concatenate([prev, y[:, 0:128]], axis=0)
    carry_ref[...] = y[t - 2:t, 0:128]

    s0 = full01[0:t, 0:64] + pb[0][None, :]
    s1 = full01[1:t + 1, 64:128] + pb[1][None, :]
    s2 = y[:, 128:192] + pb[2][None, :]

    o0 = (s0 >= s1) & (s0 >= s2)
    o1 = jnp.logical_not(o0) & (s1 >= s2)
    f0 = o0.astype(jnp.bfloat16)
    f1 = o1.astype(jnp.bfloat16)

    mem = mem_ref[...]
    d0 = (mem[0] - mem[2]).astype(jnp.bfloat16)
    d1 = (mem[1] - mem[2]).astype(jnp.bfloat16)
    base = jnp.sum(mem[2], axis=0)[None, :]

    out = jax.lax.dot_general(f0, d0, (((1,), (0,)), ((), ())),
                              preferred_element_type=jnp.float32)
    out += jax.lax.dot_general(f1, d1, (((1,), (0,)), ((), ())),
                               preferred_element_type=jnp.float32)
    out_ref[...] = out + base


def kernel(x, memory, pos_bias):
    b, l, d = x.shape
    m, n = pos_bias.shape
    w = memory.transpose(1, 0, 2).reshape(n * m, d).T
    mem_t = memory.transpose(1, 0, 2)
    pb_t = pos_bias.T
    grid = (l // _TILE,)
    out = pl.pallas_call(
        _fused,
        grid=grid,
        in_specs=[
            pl.BlockSpec((_TILE, d), lambda i: (i, 0)),
            pl.BlockSpec((d, n * m), lambda i: (0, 0)),
            pl.BlockSpec((n, m, d), lambda i: (0, 0, 0)),
            pl.BlockSpec((n, m), lambda i: (0, 0)),
        ],
        out_specs=pl.BlockSpec((_TILE, d), lambda i: (i, 0)),
        scratch_shapes=[pltpu.VMEM((2, 2 * m), jnp.float32)],
        out_shape=jax.ShapeDtypeStruct((l, d), jnp.float32),
    )(x[0], w, mem_t, pb_t)
    return out[None]
